# baseline (speedup 1.0000x reference)
import jax
import jax.numpy as jnp
from jax import lax
from jax.experimental import pallas as pl
from jax.experimental.pallas import tpu as pltpu, tpu_sc as plsc

VOCAB = 1000000
EMBED_DIM = 64
BATCH = 4096
SEQ = 200
PADDED_DIM = 128

NC = 2
NS = 16
NW = NC * NS

BI = 128
NBT = BATCH // BI
CI = 8
NCT = EMBED_DIM // CI
N_UNITS = SEQ


def _make_gather():
    mesh = plsc.VectorSubcoreMesh(core_axis_name="c", subcore_axis_name="s")

    @pl.kernel(
        out_type=jax.ShapeDtypeStruct((SEQ, NCT, NBT, CI, BI), jnp.float32),
        mesh=mesh,
        scratch_types=[
            pltpu.VMEM((SEQ, BI), jnp.int32),
            pltpu.VMEM((BI, PADDED_DIM), jnp.float32),
            pltpu.VMEM((BI, PADDED_DIM), jnp.float32),
            pltpu.VMEM((NCT, CI, BI), jnp.float32),
            pltpu.VMEM((NCT, CI, BI), jnp.float32),
            pltpu.SemaphoreType.DMA,
            pltpu.SemaphoreType.DMA,
            pltpu.SemaphoreType.DMA,
        ],
        compiler_params=pltpu.CompilerParams(
            use_tc_tiling_on_sc=False, needs_layout_passes=False),
    )
    def k(table_hbm, idx_hbm, out_hbm, idx_v, g0, g1, t0, t1, gsem, osem0, osem1):
        wid = lax.axis_index("s") * NC + lax.axis_index("c")
        pltpu.sync_copy(idx_hbm.at[wid], idx_v)

        lane = lax.iota(jnp.int32, 16)

        def gather_descr(u, gbuf):
            return pltpu.make_async_copy(table_hbm.at[idx_v.at[u]], gbuf, gsem)

        def out_descr(u, tbuf, osem):
            return pltpu.make_async_copy(tbuf, out_hbm.at[u, :, wid], osem)

        def transpose(gbuf, tbuf):
            def c_body(c, _):
                ct = c // CI
                ci = c % CI
                def g_body(g, _):
                    rows = g * 16 + lane
                    cols = jnp.full((16,), c, jnp.int32)
                    vals = plsc.load_gather(gbuf, [rows, cols])
                    tbuf[ct, ci, pl.ds(g * 16, 16)] = vals
                    return ()
                lax.fori_loop(0, BI // 16, g_body, (), unroll=True)
                return ()
            lax.fori_loop(0, EMBED_DIM, c_body, (), unroll=False)

        def step(u, gbuf, tbuf, osem, first):
            gather_descr(u, gbuf).wait()
            if not first:
                out_descr(u - 2, tbuf, osem).wait()
            transpose(gbuf, tbuf)
            out_descr(u, tbuf, osem).start()

        gather_descr(0, g0).start()
        gather_descr(1, g1).start()

        step(0, g0, t0, osem0, True)
        gather_descr(2, g0).start()
        step(1, g1, t1, osem1, True)
        gather_descr(3, g1).start()

        def body(i, _):
            u0 = 2 * i
            step(u0, g0, t0, osem0, False)
            gather_descr(u0 + 2, g0).start()
            u1 = u0 + 1
            step(u1, g1, t1, osem1, False)
            gather_descr(u1 + 2, g1).start()
            return ()

        lax.fori_loop(1, N_UNITS // 2 - 1, body, (), unroll=False)

        step(N_UNITS - 2, g0, t0, osem0, False)
        step(N_UNITS - 1, g1, t1, osem1, False)
        out_descr(N_UNITS - 2, t0, osem0).wait()
        out_descr(N_UNITS - 1, t1, osem1).wait()

    return k


_gather = _make_gather()


def kernel(tokens, embed_weight):
    table = jnp.pad(embed_weight, ((0, 0), (0, PADDED_DIM - EMBED_DIM)))
    idx = tokens.astype(jnp.int32).T.reshape(SEQ, NBT, BI).transpose(1, 0, 2)
    out5 = _gather(table, idx)
    return out5.transpose(2, 4, 0, 1, 3).reshape(BATCH, SEQ, EMBED_DIM)
